# baseline (speedup 1.0000x reference)
import jax
import jax.numpy as jnp
from jax import lax
from jax.experimental import pallas as pl
from jax.experimental.pallas import tpu as pltpu
from jax.experimental.pallas import tpu_sc as plsc

T = 2048
D = 1024
H = 4096
E = 8
HR = 2048
TILE = 256
NT = 23
ROWS = NT * TILE
NC = 2
NS = 16
NW = NC * NS
TPW = T // NW
APS = (2 * T) // NS

_f32 = jnp.float32
_i32 = jnp.int32


def _sc_mesh():
    return plsc.VectorSubcoreMesh(
        core_axis_name="c", subcore_axis_name="s", num_cores=NC,
        num_subcores=NS)


def _router_body(x_ref, wg_ref, gum_ref, idx_ref, wts_ref, loss_ref):
    x = x_ref[...]
    wg = wg_ref[...]
    logits = lax.dot_general(x, wg, (((1,), (1,)), ((), ())),
                             preferred_element_type=_f32)
    logits = logits + gum_ref[...]
    m = jnp.max(logits, axis=-1, keepdims=True)
    ex = jnp.exp(logits - m)
    p = ex / jnp.sum(ex, axis=-1, keepdims=True)
    me = jnp.mean(p, axis=0)
    ce = jnp.mean(p * p, axis=0)
    loss_ref[0, 0] = jnp.sum(me * ce) * float(E * E)
    a1 = jnp.argmax(p, axis=-1).astype(_i32)
    w1 = jnp.max(p, axis=-1)
    eiota = lax.broadcasted_iota(_i32, (T, E), 1)
    p2 = jnp.where(eiota == a1[:, None], -jnp.inf, p)
    a2 = jnp.argmax(p2, axis=-1).astype(_i32)
    w2 = jnp.max(p2, axis=-1)
    idx_ref[0:1, :] = a1[None, :]
    idx_ref[1:2, :] = a2[None, :]
    wts_ref[0:1, :] = w1[None, :]
    wts_ref[1:2, :] = w2[None, :]


def _run_router(xs, w_gating, gum):
    return pl.pallas_call(
        _router_body,
        out_shape=(
            jax.ShapeDtypeStruct((2, T), _i32),
            jax.ShapeDtypeStruct((2, T), _f32),
            jax.ShapeDtypeStruct((1, 1), _f32),
        ),
        out_specs=(
            pl.BlockSpec(memory_space=pltpu.VMEM),
            pl.BlockSpec(memory_space=pltpu.VMEM),
            pl.BlockSpec(memory_space=pltpu.SMEM),
        ),
    )(xs, w_gating, gum)


def _meta_body(idx_hbm, wts_hbm, dests_hbm, wbuf_hbm, eot_hbm,
               e_all, d2_ref, w_ref, wrow_ref, run_ref, eotv_ref, sem):
    cid = lax.axis_index("c")
    sid = lax.axis_index("s")
    wid = sid * NC + cid
    lanes = lax.iota(_i32, 16)
    pltpu.sync_copy(idx_hbm.at[0], e_all.at[pl.ds(0, T)])
    pltpu.sync_copy(idx_hbm.at[1], e_all.at[pl.ds(T, T)])

    APW = (2 * T) // NW
    myvec0 = wid * (APW // 16)

    def hcount(i, carry):
        tot, pref = carry
        ev = e_all[pl.ds(i * 16, 16)]
        for b in range(E):
            cnt = plsc.all_reduce_population_count(ev == b)
            sel = lanes == b
            tot = jnp.where(sel, tot + cnt, tot)
            pref = jnp.where(sel & (i < myvec0), pref + cnt, pref)
        return tot, pref
    tot, pref = lax.fori_loop(0, (2 * T) // 16, hcount,
                              (jnp.zeros((16,), _i32),
                               jnp.zeros((16,), _i32)))

    padded = ((tot + (TILE - 1)) >> 8) << 8
    pbase = plsc.cumsum(padded) - padded
    my_start = pbase + pref
    te = plsc.cumsum(padded >> 8)

    @pl.when(wid == 0)
    def _eot():
        for half in range(2):
            tv = lax.iota(_i32, 16) + 16 * half
            cnt = jnp.zeros((16,), _i32)
            for b in range(E):
                cnt = cnt + jnp.where(tv >= te[b], 1, 0)
            eotv_ref[pl.ds(half * 16, 16)] = jnp.minimum(cnt, E - 1)
        pltpu.sync_copy(eotv_ref, eot_hbm)

    def vec_step(i, run_v):
        run_ref[...] = run_v
        ev = e_all[pl.ds((myvec0 + i) * 16, 16)]
        pc = jnp.zeros((16,), _i32)
        newrun = run_v
        for b in range(E):
            msk = ev == b
            ones = jnp.where(msk, 1, 0)
            pc = jnp.where(msk, plsc.cumsum(ones) - 1, pc)
            cnt = plsc.all_reduce_population_count(msk)
            newrun = jnp.where(lanes == b, newrun + cnt, newrun)
        dst = plsc.load_gather(run_ref, [ev]) + pc
        d2_ref[0, pl.ds(i * 16, 16)] = dst
        return newrun
    lax.fori_loop(0, APW // 16, vec_step, my_start)

    k = wid // 16
    tok0 = (wid % 16) * APW
    pltpu.sync_copy(d2_ref.at[0], dests_hbm.at[k, pl.ds(tok0, APW)])

    pltpu.sync_copy(wts_hbm.at[k, pl.ds(tok0, APW)], w_ref)

    def wrow_step(r, carry):
        ridx = jnp.zeros((16,), _i32) + r
        wrow_ref[r, pl.ds(0, 16)] = plsc.load_gather(w_ref, [ridx])
        return carry
    lax.fori_loop(0, APW, wrow_step, 0)
    pltpu.async_copy(wrow_ref, wbuf_hbm.at[d2_ref.at[0]], sem).wait()


def _run_meta(idx, wts):
    kern = pl.kernel(
        _meta_body,
        out_type=(
            jax.ShapeDtypeStruct((2, T), _i32),
            jax.ShapeDtypeStruct((ROWS, 128), _f32),
            jax.ShapeDtypeStruct((32,), _i32),
        ),
        mesh=_sc_mesh(),
        scratch_types=[
            pltpu.VMEM((2 * T,), _i32),
            pltpu.VMEM((1, 128), _i32),
            pltpu.VMEM((128,), _f32),
            pltpu.VMEM((128, 128), _f32),
            pltpu.VMEM((16,), _i32),
            pltpu.VMEM((32,), _i32),
            pltpu.SemaphoreType.DMA,
        ],
        compiler_params=pltpu.CompilerParams(needs_layout_passes=False),
    )
    return kern(idx, wts)


def _dispatch_body(x_hbm, dests_hbm, grouped_hbm, xr_ref, i0_ref, i1_ref,
                   sem0, sem1):
    wid = lax.axis_index("s") * NC + lax.axis_index("c")
    tok0 = wid * TPW
    pltpu.sync_copy(x_hbm.at[pl.ds(tok0, TPW)], xr_ref)
    pltpu.sync_copy(dests_hbm.at[0, pl.ds(tok0, TPW)], i0_ref)
    pltpu.sync_copy(dests_hbm.at[1, pl.ds(tok0, TPW)], i1_ref)
    c0 = pltpu.async_copy(xr_ref, grouped_hbm.at[i0_ref], sem0)
    c1 = pltpu.async_copy(xr_ref, grouped_hbm.at[i1_ref], sem1)
    c0.wait()
    c1.wait()


def _run_dispatch(xs, dests):
    kern = pl.kernel(
        _dispatch_body,
        out_type=jax.ShapeDtypeStruct((ROWS, D), _f32),
        mesh=_sc_mesh(),
        scratch_types=[
            pltpu.VMEM((TPW, D), _f32),
            pltpu.VMEM((TPW,), _i32),
            pltpu.VMEM((TPW,), _i32),
            pltpu.SemaphoreType.DMA,
            pltpu.SemaphoreType.DMA,
        ],
        compiler_params=pltpu.CompilerParams(needs_layout_passes=False),
    )
    return kern(xs, dests)


def _grouped_body(eot_ref, xr_ref, wr_ref, w1_ref, b1_ref, w2_ref, b2_ref,
                  o_ref, w1bf_ref, w2bf_ref):
    i = pl.program_id(0)
    first = i == 0
    changed = eot_ref[i] != eot_ref[jnp.maximum(i - 1, 0)]

    @pl.when(first | changed)
    def _cast():
        w1bf_ref[...] = w1_ref[0].astype(jnp.bfloat16)
        w2bf_ref[...] = w2_ref[0].astype(jnp.bfloat16)

    rows = xr_ref[...].astype(jnp.bfloat16)
    h = lax.dot_general(rows, w1bf_ref[...], (((1,), (1,)), ((), ())),
                        preferred_element_type=_f32)
    h = jnp.maximum(h + b1_ref[0], 0.0).astype(jnp.bfloat16)
    o = lax.dot_general(h, w2bf_ref[...], (((1,), (1,)), ((), ())),
                        preferred_element_type=_f32)
    o_ref[...] = (o + b2_ref[0]) * wr_ref[:, 0:1]


def _run_grouped(eot, grouped, wbuf, W1r, b1r, W2r, b2r):
    grid_spec = pltpu.PrefetchScalarGridSpec(
        num_scalar_prefetch=1,
        grid=(NT,),
        in_specs=[
            pl.BlockSpec((TILE, D), lambda i, eot: (i, 0)),
            pl.BlockSpec((TILE, 128), lambda i, eot: (i, 0)),
            pl.BlockSpec((1, HR, D), lambda i, eot: (eot[i], 0, 0)),
            pl.BlockSpec((1, 1, HR), lambda i, eot: (eot[i], 0, 0)),
            pl.BlockSpec((1, D, HR), lambda i, eot: (eot[i], 0, 0)),
            pl.BlockSpec((1, 1, D), lambda i, eot: (eot[i], 0, 0)),
        ],
        out_specs=pl.BlockSpec((TILE, D), lambda i, eot: (i, 0)),
        scratch_shapes=[
            pltpu.VMEM((HR, D), jnp.bfloat16),
            pltpu.VMEM((D, HR), jnp.bfloat16),
        ],
    )
    return pl.pallas_call(
        _grouped_body,
        grid_spec=grid_spec,
        out_shape=jax.ShapeDtypeStruct((ROWS, D), _f32),
        compiler_params=pltpu.CompilerParams(
            dimension_semantics=("arbitrary",)),
    )(eot, grouped, wbuf, W1r, b1r, W2r, b2r)


def _shared_body(x_ref, w1_ref, b1_ref, w2_ref, b2_ref, o_ref):
    hc = pl.program_id(0)
    h = lax.dot_general(x_ref[...].astype(jnp.bfloat16),
                        w1_ref[...].astype(jnp.bfloat16),
                        (((1,), (1,)), ((), ())),
                        preferred_element_type=_f32)
    h = jnp.maximum(h + b1_ref[...], 0.0).astype(jnp.bfloat16)
    part = lax.dot_general(h, w2_ref[...].astype(jnp.bfloat16),
                           (((1,), (1,)), ((), ())),
                           preferred_element_type=_f32)

    @pl.when(hc == 0)
    def _first():
        o_ref[...] = part + b2_ref[...]

    @pl.when(hc != 0)
    def _rest():
        o_ref[...] = o_ref[...] + part


def _run_shared(xs, W1s, b1s2, W2s, b2s2):
    HC = 512
    return pl.pallas_call(
        _shared_body,
        grid=(H // HC,),
        in_specs=[
            pl.BlockSpec((T, D), lambda hc: (0, 0)),
            pl.BlockSpec((HC, D), lambda hc: (hc, 0)),
            pl.BlockSpec((1, HC), lambda hc: (0, hc)),
            pl.BlockSpec((D, HC), lambda hc: (0, hc)),
            pl.BlockSpec((1, D), lambda hc: (0, 0)),
        ],
        out_specs=pl.BlockSpec((T, D), lambda hc: (0, 0)),
        out_shape=jax.ShapeDtypeStruct((T, D), _f32),
        compiler_params=pltpu.CompilerParams(
            dimension_semantics=("arbitrary",)),
    )(xs, W1s, b1s2, W2s, b2s2)


def _combine_body(sh_hbm, eo_hbm, dests_hbm, out_hbm,
                  i0_ref, i1_ref, r0_ref, r1_ref, s_ref, sem0, sem1):
    wid = lax.axis_index("s") * NC + lax.axis_index("c")
    tok0 = wid * TPW
    CH = TPW // 2
    pltpu.sync_copy(dests_hbm.at[0, pl.ds(tok0, TPW)], i0_ref)
    pltpu.sync_copy(dests_hbm.at[1, pl.ds(tok0, TPW)], i1_ref)
    for c in range(2):
        g0 = pltpu.async_copy(eo_hbm.at[i0_ref.at[pl.ds(c * CH, CH)]],
                              r0_ref, sem0)
        g1 = pltpu.async_copy(eo_hbm.at[i1_ref.at[pl.ds(c * CH, CH)]],
                              r1_ref, sem1)
        pltpu.sync_copy(sh_hbm.at[pl.ds(tok0 + c * CH, CH)], s_ref)
        g0.wait()
        g1.wait()

        def row_step(r, carry):
            def col_step(cc, carry2):
                for u in range(4):
                    sl = pl.ds((cc * 4 + u) * 16, 16)
                    r0_ref[r, sl] = (r0_ref[r, sl] + r1_ref[r, sl]
                                     + s_ref[r, sl])
                return carry2
            return lax.fori_loop(0, D // 64, col_step, carry)
        lax.fori_loop(0, CH, row_step, 0)
        pltpu.sync_copy(r0_ref, out_hbm.at[pl.ds(tok0 + c * CH, CH)])


def _run_combine(sh, eo, dests):
    CH = TPW // 2
    kern = pl.kernel(
        _combine_body,
        out_type=jax.ShapeDtypeStruct((T, D), _f32),
        mesh=_sc_mesh(),
        scratch_types=[
            pltpu.VMEM((TPW,), _i32),
            pltpu.VMEM((TPW,), _i32),
            pltpu.VMEM((CH, D), _f32),
            pltpu.VMEM((CH, D), _f32),
            pltpu.VMEM((CH, D), _f32),
            pltpu.SemaphoreType.DMA,
            pltpu.SemaphoreType.DMA,
        ],
        compiler_params=pltpu.CompilerParams(needs_layout_passes=False),
    )
    return kern(sh, eo, dests)


def kernel(x, w_gating, W1_shared, b1_shared, W2_shared, b2_shared,
           W1_routed, b1_routed, W2_routed, b2_routed):
    xs = x.reshape(T, D)
    noise = jax.random.uniform(jax.random.key(42), (1, T, E), dtype=_f32)
    gum = (-jnp.log(-jnp.log(noise + 1e-9) + 1e-9)).reshape(T, E)

    idx, wts, loss = _run_router(xs, w_gating, gum)
    dests, wbuf, eot = _run_meta(idx, wts)
    grouped = _run_dispatch(xs, dests)
    eo = _run_grouped(eot, grouped, wbuf, W1_routed,
                      b1_routed.reshape(E, 1, HR), W2_routed,
                      b2_routed.reshape(E, 1, D))
    sh = _run_shared(xs, W1_shared, b1_shared.reshape(1, H), W2_shared,
                     b2_shared.reshape(1, D))
    out = _run_combine(sh, eo, dests)
    return out.reshape(1, T, D), loss[0, 0]


# speedup vs baseline: 1.0296x; 1.0296x over previous
import jax
import jax.numpy as jnp
from jax import lax
from jax.experimental import pallas as pl
from jax.experimental.pallas import tpu as pltpu
from jax.experimental.pallas import tpu_sc as plsc

T = 2048
D = 1024
H = 4096
E = 8
HR = 2048
TILE = 256
NT = 23
ROWS = NT * TILE
NC = 2
NS = 16
NW = NC * NS
TPW = T // NW
APS = (2 * T) // NS

_f32 = jnp.float32
_i32 = jnp.int32


def _sc_mesh():
    return plsc.VectorSubcoreMesh(
        core_axis_name="c", subcore_axis_name="s", num_cores=NC,
        num_subcores=NS)


def _router_body(x_ref, wg_ref, gum_ref, idx_ref, wts_ref, loss_ref):
    x = x_ref[...]
    wg = wg_ref[...]
    logits = lax.dot_general(x, wg, (((1,), (1,)), ((), ())),
                             preferred_element_type=_f32)
    logits = logits + gum_ref[...]
    m = jnp.max(logits, axis=-1, keepdims=True)
    ex = jnp.exp(logits - m)
    p = ex / jnp.sum(ex, axis=-1, keepdims=True)
    me = jnp.mean(p, axis=0)
    ce = jnp.mean(p * p, axis=0)
    loss_ref[0, 0] = jnp.sum(me * ce) * float(E * E)
    a1 = jnp.argmax(p, axis=-1).astype(_i32)
    w1 = jnp.max(p, axis=-1)
    eiota = lax.broadcasted_iota(_i32, (T, E), 1)
    p2 = jnp.where(eiota == a1[:, None], -jnp.inf, p)
    a2 = jnp.argmax(p2, axis=-1).astype(_i32)
    w2 = jnp.max(p2, axis=-1)
    idx_ref[0:1, :] = a1[None, :]
    idx_ref[1:2, :] = a2[None, :]
    wts_ref[0:1, :] = w1[None, :]
    wts_ref[1:2, :] = w2[None, :]


def _run_router(xs, w_gating, gum):
    return pl.pallas_call(
        _router_body,
        out_shape=(
            jax.ShapeDtypeStruct((2, T), _i32),
            jax.ShapeDtypeStruct((2, T), _f32),
            jax.ShapeDtypeStruct((1, 1), _f32),
        ),
        out_specs=(
            pl.BlockSpec(memory_space=pltpu.VMEM),
            pl.BlockSpec(memory_space=pltpu.VMEM),
            pl.BlockSpec(memory_space=pltpu.SMEM),
        ),
    )(xs, w_gating, gum)


def _meta_body(idx_hbm, wts_hbm, dests_hbm, wbuf_hbm, eot_hbm,
               e_all, d2_ref, w_ref, wrow_ref, run_ref, eotv_ref, sem):
    cid = lax.axis_index("c")
    sid = lax.axis_index("s")
    wid = sid * NC + cid
    lanes = lax.iota(_i32, 16)
    pltpu.sync_copy(idx_hbm.at[0], e_all.at[pl.ds(0, T)])
    pltpu.sync_copy(idx_hbm.at[1], e_all.at[pl.ds(T, T)])

    APW = (2 * T) // NW
    myvec0 = wid * (APW // 16)

    def hcount(i, carry):
        tot, pref = carry
        ev = e_all[pl.ds(i * 16, 16)]
        for b in range(E):
            cnt = plsc.all_reduce_population_count(ev == b)
            sel = lanes == b
            tot = jnp.where(sel, tot + cnt, tot)
            pref = jnp.where(sel & (i < myvec0), pref + cnt, pref)
        return tot, pref
    tot, pref = lax.fori_loop(0, (2 * T) // 16, hcount,
                              (jnp.zeros((16,), _i32),
                               jnp.zeros((16,), _i32)))

    padded = ((tot + (TILE - 1)) >> 8) << 8
    pbase = plsc.cumsum(padded) - padded
    my_start = pbase + pref
    te = plsc.cumsum(padded >> 8)

    @pl.when(wid == 0)
    def _eot():
        for half in range(2):
            tv = lax.iota(_i32, 16) + 16 * half
            cnt = jnp.zeros((16,), _i32)
            for b in range(E):
                cnt = cnt + jnp.where(tv >= te[b], 1, 0)
            eotv_ref[pl.ds(half * 16, 16)] = jnp.minimum(cnt, E - 1)
        pltpu.sync_copy(eotv_ref, eot_hbm)

    def vec_step(i, run_v):
        run_ref[...] = run_v
        ev = e_all[pl.ds((myvec0 + i) * 16, 16)]
        pc = jnp.zeros((16,), _i32)
        newrun = run_v
        for b in range(E):
            msk = ev == b
            ones = jnp.where(msk, 1, 0)
            pc = jnp.where(msk, plsc.cumsum(ones) - 1, pc)
            cnt = plsc.all_reduce_population_count(msk)
            newrun = jnp.where(lanes == b, newrun + cnt, newrun)
        dst = plsc.load_gather(run_ref, [ev]) + pc
        d2_ref[0, pl.ds(i * 16, 16)] = dst
        return newrun
    lax.fori_loop(0, APW // 16, vec_step, my_start)

    k = wid // 16
    tok0 = (wid % 16) * APW
    pltpu.sync_copy(d2_ref.at[0], dests_hbm.at[k, pl.ds(tok0, APW)])

    pltpu.sync_copy(wts_hbm.at[k, pl.ds(tok0, APW)], w_ref)

    def wrow_step(r, carry):
        ridx = jnp.zeros((16,), _i32) + r
        wrow_ref[r, pl.ds(0, 16)] = plsc.load_gather(w_ref, [ridx])
        return carry
    lax.fori_loop(0, APW, wrow_step, 0)
    pltpu.async_copy(wrow_ref, wbuf_hbm.at[d2_ref.at[0]], sem).wait()


def _run_meta(idx, wts):
    kern = pl.kernel(
        _meta_body,
        out_type=(
            jax.ShapeDtypeStruct((2, T), _i32),
            jax.ShapeDtypeStruct((ROWS, 128), _f32),
            jax.ShapeDtypeStruct((32,), _i32),
        ),
        mesh=_sc_mesh(),
        scratch_types=[
            pltpu.VMEM((2 * T,), _i32),
            pltpu.VMEM((1, 128), _i32),
            pltpu.VMEM((128,), _f32),
            pltpu.VMEM((128, 128), _f32),
            pltpu.VMEM((16,), _i32),
            pltpu.VMEM((32,), _i32),
            pltpu.SemaphoreType.DMA,
        ],
        compiler_params=pltpu.CompilerParams(needs_layout_passes=False),
    )
    return kern(idx, wts)


def _dispatch_body(x_hbm, dests_hbm, grouped_hbm, xr_ref, i0_ref, i1_ref,
                   sem0, sem1):
    wid = lax.axis_index("s") * NC + lax.axis_index("c")
    tok0 = wid * TPW
    pltpu.sync_copy(x_hbm.at[pl.ds(tok0, TPW)], xr_ref)
    pltpu.sync_copy(dests_hbm.at[0, pl.ds(tok0, TPW)], i0_ref)
    pltpu.sync_copy(dests_hbm.at[1, pl.ds(tok0, TPW)], i1_ref)
    c0 = pltpu.async_copy(xr_ref, grouped_hbm.at[i0_ref], sem0)
    c1 = pltpu.async_copy(xr_ref, grouped_hbm.at[i1_ref], sem1)
    c0.wait()
    c1.wait()


def _run_dispatch(xs, dests):
    kern = pl.kernel(
        _dispatch_body,
        out_type=jax.ShapeDtypeStruct((ROWS, D), _f32),
        mesh=_sc_mesh(),
        scratch_types=[
            pltpu.VMEM((TPW, D), _f32),
            pltpu.VMEM((TPW,), _i32),
            pltpu.VMEM((TPW,), _i32),
            pltpu.SemaphoreType.DMA,
            pltpu.SemaphoreType.DMA,
        ],
        compiler_params=pltpu.CompilerParams(needs_layout_passes=False),
    )
    return kern(xs, dests)


def _grouped_body(eot_ref, xr_ref, wr_ref, w1_ref, b1_ref, w2_ref, b2_ref,
                  o_ref):
    rows = xr_ref[...].astype(jnp.bfloat16)
    h = lax.dot_general(rows, w1_ref[0].astype(jnp.bfloat16),
                        (((1,), (1,)), ((), ())),
                        preferred_element_type=_f32)
    h = jnp.maximum(h + b1_ref[0], 0.0).astype(jnp.bfloat16)
    o = lax.dot_general(h, w2_ref[0].astype(jnp.bfloat16),
                        (((1,), (1,)), ((), ())),
                        preferred_element_type=_f32)
    o_ref[...] = (o + b2_ref[0]) * wr_ref[:, 0:1]


def _run_grouped(eot, grouped, wbuf, W1r, b1r, W2r, b2r):
    grid_spec = pltpu.PrefetchScalarGridSpec(
        num_scalar_prefetch=1,
        grid=(NT,),
        in_specs=[
            pl.BlockSpec((TILE, D), lambda i, eot: (i, 0)),
            pl.BlockSpec((TILE, 128), lambda i, eot: (i, 0)),
            pl.BlockSpec((1, HR, D), lambda i, eot: (eot[i], 0, 0)),
            pl.BlockSpec((1, 1, HR), lambda i, eot: (eot[i], 0, 0)),
            pl.BlockSpec((1, D, HR), lambda i, eot: (eot[i], 0, 0)),
            pl.BlockSpec((1, 1, D), lambda i, eot: (eot[i], 0, 0)),
        ],
        out_specs=pl.BlockSpec((TILE, D), lambda i, eot: (i, 0)),
    )
    return pl.pallas_call(
        _grouped_body,
        grid_spec=grid_spec,
        out_shape=jax.ShapeDtypeStruct((ROWS, D), _f32),
        compiler_params=pltpu.CompilerParams(
            dimension_semantics=("arbitrary",)),
    )(eot, grouped, wbuf, W1r, b1r, W2r, b2r)


def _shared_body(x_ref, w1_ref, b1_ref, w2_ref, b2_ref, o_ref):
    hc = pl.program_id(0)
    h = lax.dot_general(x_ref[...].astype(jnp.bfloat16),
                        w1_ref[...].astype(jnp.bfloat16),
                        (((1,), (1,)), ((), ())),
                        preferred_element_type=_f32)
    h = jnp.maximum(h + b1_ref[...], 0.0).astype(jnp.bfloat16)
    part = lax.dot_general(h, w2_ref[...].astype(jnp.bfloat16),
                           (((1,), (1,)), ((), ())),
                           preferred_element_type=_f32)

    @pl.when(hc == 0)
    def _first():
        o_ref[...] = part + b2_ref[...]

    @pl.when(hc != 0)
    def _rest():
        o_ref[...] = o_ref[...] + part


def _run_shared(xs, W1s, b1s2, W2s, b2s2):
    HC = 512
    return pl.pallas_call(
        _shared_body,
        grid=(H // HC,),
        in_specs=[
            pl.BlockSpec((T, D), lambda hc: (0, 0)),
            pl.BlockSpec((HC, D), lambda hc: (hc, 0)),
            pl.BlockSpec((1, HC), lambda hc: (0, hc)),
            pl.BlockSpec((D, HC), lambda hc: (0, hc)),
            pl.BlockSpec((1, D), lambda hc: (0, 0)),
        ],
        out_specs=pl.BlockSpec((T, D), lambda hc: (0, 0)),
        out_shape=jax.ShapeDtypeStruct((T, D), _f32),
        compiler_params=pltpu.CompilerParams(
            dimension_semantics=("arbitrary",)),
    )(xs, W1s, b1s2, W2s, b2s2)


def _combine_body(sh_hbm, eo_hbm, dests_hbm, out_hbm,
                  i0_ref, i1_ref, r0_ref, r1_ref, s_ref, sem0, sem1):
    wid = lax.axis_index("s") * NC + lax.axis_index("c")
    tok0 = wid * TPW
    CH = TPW // 2
    pltpu.sync_copy(dests_hbm.at[0, pl.ds(tok0, TPW)], i0_ref)
    pltpu.sync_copy(dests_hbm.at[1, pl.ds(tok0, TPW)], i1_ref)
    for c in range(2):
        g0 = pltpu.async_copy(eo_hbm.at[i0_ref.at[pl.ds(c * CH, CH)]],
                              r0_ref, sem0)
        g1 = pltpu.async_copy(eo_hbm.at[i1_ref.at[pl.ds(c * CH, CH)]],
                              r1_ref, sem1)
        pltpu.sync_copy(sh_hbm.at[pl.ds(tok0 + c * CH, CH)], s_ref)
        g0.wait()
        g1.wait()

        def row_step(r, carry):
            def col_step(cc, carry2):
                for u in range(4):
                    sl = pl.ds((cc * 4 + u) * 16, 16)
                    r0_ref[r, sl] = (r0_ref[r, sl] + r1_ref[r, sl]
                                     + s_ref[r, sl])
                return carry2
            return lax.fori_loop(0, D // 64, col_step, carry)
        lax.fori_loop(0, CH, row_step, 0)
        pltpu.sync_copy(r0_ref, out_hbm.at[pl.ds(tok0 + c * CH, CH)])


def _run_combine(sh, eo, dests):
    CH = TPW // 2
    kern = pl.kernel(
        _combine_body,
        out_type=jax.ShapeDtypeStruct((T, D), _f32),
        mesh=_sc_mesh(),
        scratch_types=[
            pltpu.VMEM((TPW,), _i32),
            pltpu.VMEM((TPW,), _i32),
            pltpu.VMEM((CH, D), _f32),
            pltpu.VMEM((CH, D), _f32),
            pltpu.VMEM((CH, D), _f32),
            pltpu.SemaphoreType.DMA,
            pltpu.SemaphoreType.DMA,
        ],
        compiler_params=pltpu.CompilerParams(needs_layout_passes=False),
    )
    return kern(sh, eo, dests)


def kernel(x, w_gating, W1_shared, b1_shared, W2_shared, b2_shared,
           W1_routed, b1_routed, W2_routed, b2_routed):
    xs = x.reshape(T, D)
    noise = jax.random.uniform(jax.random.key(42), (1, T, E), dtype=_f32)
    gum = (-jnp.log(-jnp.log(noise + 1e-9) + 1e-9)).reshape(T, E)

    idx, wts, loss = _run_router(xs, w_gating, gum)
    dests, wbuf, eot = _run_meta(idx, wts)
    grouped = _run_dispatch(xs, dests)
    sh = _run_shared(xs, W1_shared, b1_shared.reshape(1, H), W2_shared,
                     b2_shared.reshape(1, D))
    eo = _run_grouped(eot, grouped, wbuf, W1_routed,
                      b1_routed.reshape(E, 1, HR), W2_routed,
                      b2_routed.reshape(E, 1, D))
    out = _run_combine(sh, eo, dests)
    return out.reshape(1, T, D), loss[0, 0]
